# Initial kernel scaffold; baseline (speedup 1.0000x reference)
import functools

import jax
import jax.numpy as jnp
from jax import lax
from jax.experimental import pallas as pl
from jax.experimental.pallas import tpu as pltpu
from jax.experimental.pallas import tpu_sc as plsc

USER_N = 50000
ITEM_N = 50000
NN = USER_N + ITEM_N
DD = 32
HALF = 16
NC = 2
NS = 16
LANES = 16

CHUNK = 128
SUB = 16
BLOCK = CHUNK * SUB
ROWS_PER_TILE = NN // NS
ZR = 250

MEAN_CH = 4000
TOT = NN * DD
PER_W = TOT // (NC * NS)


@functools.lru_cache(maxsize=None)
def _layer_fn(nblk):
    mesh = plsc.VectorSubcoreMesh(core_axis_name="c", subcore_axis_name="s")

    def body(table, src, dst2d, w, out, acc, zbuf, src_b, gidx_b, dst_b,
             w_b, rows, gsem):
        c = lax.axis_index("c")
        s = lax.axis_index("s")

        zero16 = jnp.zeros((LANES,), jnp.float32)
        for i in range(ZR):
            zbuf[i, 0, :] = zero16
        base_row = s * ROWS_PER_TILE

        def zero_body(k, carry):
            pltpu.sync_copy(zbuf, acc.at[pl.ds(base_row + k * ZR, ZR)])
            return carry

        lax.fori_loop(0, ROWS_PER_TILE // ZR, zero_body, 0)
        plsc.subcore_barrier()

        ebase = s * (nblk * BLOCK)
        cbase = s * (nblk * SUB)

        def block_body(b, carry):
            eoff = ebase + b * BLOCK
            pltpu.sync_copy(src.at[pl.ds(eoff, BLOCK)], src_b)
            pltpu.sync_copy(w.at[pl.ds(eoff, BLOCK)], w_b)
            pltpu.sync_copy(dst2d.at[pl.ds(cbase + b * SUB, SUB)], dst_b)

            for q in range(BLOCK // LANES):
                sv = src_b[pl.ds(q * LANES, LANES)]
                gidx_b[q // (CHUNK // LANES),
                       pl.ds((q % (CHUNK // LANES)) * LANES, LANES)] = (
                           sv * 2 + c)

            def sub_body(j, icarry):
                pltpu.async_copy(table.at[gidx_b.at[j]], rows, gsem).wait()
                for q in range(CHUNK // LANES):
                    w16 = w_b[pl.ds(j * CHUNK + q * LANES, LANES)]
                    for e in range(LANES):
                        r = q * LANES + e
                        wv = lax.broadcast(w16[e], (LANES,))
                        rows[r, 0, :] = rows[r, 0, :] * wv
                pltpu.sync_copy(rows, acc.at[dst_b.at[j]], add=True)
                return icarry

            lax.fori_loop(0, SUB, sub_body, 0)
            return carry

        lax.fori_loop(0, nblk, block_body, 0)
        plsc.subcore_barrier()

        pltpu.sync_copy(
            acc.at[pl.ds(base_row, ROWS_PER_TILE)],
            out.at[pl.ds(base_row, ROWS_PER_TILE), pl.ds(c, 1)])

    return pl.kernel(
        body,
        out_type=jax.ShapeDtypeStruct((NN, NC, HALF), jnp.float32),
        mesh=mesh,
        scratch_types=[
            pltpu.VMEM_SHARED((NN, 1, HALF), jnp.float32),
            pltpu.VMEM((ZR, 1, HALF), jnp.float32),
            pltpu.VMEM((BLOCK,), jnp.int32),
            pltpu.VMEM((SUB, CHUNK), jnp.int32),
            pltpu.VMEM((SUB, CHUNK), jnp.int32),
            pltpu.VMEM((BLOCK,), jnp.float32),
            pltpu.VMEM((CHUNK, 1, HALF), jnp.float32),
            pltpu.SemaphoreType.DMA,
        ],
    )


def _mean_body(e0, e1, e2, e3, out, b0, b1, b2, b3, ob):
    c = lax.axis_index("c")
    s = lax.axis_index("s")
    wid = s * NC + c
    base = wid * PER_W

    def chunk_body(k, carry):
        off = base + k * MEAN_CH
        pltpu.sync_copy(e0.at[pl.ds(off, MEAN_CH)], b0)
        pltpu.sync_copy(e1.at[pl.ds(off, MEAN_CH)], b1)
        pltpu.sync_copy(e2.at[pl.ds(off, MEAN_CH)], b2)
        pltpu.sync_copy(e3.at[pl.ds(off, MEAN_CH)], b3)
        for q in range(MEAN_CH // LANES):
            sl = pl.ds(q * LANES, LANES)
            ob[sl] = (b0[sl] + b1[sl] + b2[sl] + b3[sl]) * 0.25
        pltpu.sync_copy(ob, out.at[pl.ds(off, MEAN_CH)])
        return carry

    lax.fori_loop(0, PER_W // MEAN_CH, chunk_body, 0)


_mean_fn = pl.kernel(
    _mean_body,
    out_type=jax.ShapeDtypeStruct((TOT,), jnp.float32),
    mesh=plsc.VectorSubcoreMesh(core_axis_name="c", subcore_axis_name="s"),
    scratch_types=[pltpu.VMEM((MEAN_CH,), jnp.float32) for _ in range(5)],
)


def kernel(user_emb, item_emb, edge_index, edge_weight):
    ego0 = jnp.concatenate([user_emb, item_emb], axis=0)
    src = edge_index[0]
    dst = edge_index[1]
    e = src.shape[0]
    nblk = -(-e // (NS * BLOCK))
    epad = nblk * NS * BLOCK
    pad = epad - e
    src_p = jnp.concatenate([src, jnp.zeros((pad,), jnp.int32)])
    dst_p = jnp.concatenate([dst, jnp.zeros((pad,), jnp.int32)])
    w_p = jnp.concatenate([edge_weight, jnp.zeros((pad,), jnp.float32)])
    dst2d = dst_p.reshape(-1, CHUNK)

    layer = _layer_fn(nblk)
    t0 = ego0.reshape(NN * 2, 1, HALF)
    t1 = layer(t0, src_p, dst2d, w_p)
    t2 = layer(t1.reshape(NN * 2, 1, HALF), src_p, dst2d, w_p)
    t3 = layer(t2.reshape(NN * 2, 1, HALF), src_p, dst2d, w_p)

    m = _mean_fn(ego0.reshape(TOT), t1.reshape(TOT),
                 t2.reshape(TOT), t3.reshape(TOT))
    m = m.reshape(NN, DD)
    return (m[:USER_N], m[USER_N:])


# speedup vs baseline: 6.2276x; 6.2276x over previous
import functools

import jax
import jax.numpy as jnp
from jax import lax
from jax.experimental import pallas as pl
from jax.experimental.pallas import tpu as pltpu
from jax.experimental.pallas import tpu_sc as plsc

USER_N = 50000
ITEM_N = 50000
NN = USER_N + ITEM_N
DD = 32
HALF = 16
NC = 2
NS = 16
LANES = 16

CHUNK = 128
SUB = 16
BLOCK = CHUNK * SUB
ROWS_PER_TILE = NN // NS
ZR = 250

MEAN_CH = 4000
TOT = NN * DD
PER_W = TOT // (NC * NS)


@functools.lru_cache(maxsize=None)
def _layer_fn(nblk):
    mesh = plsc.VectorSubcoreMesh(core_axis_name="c", subcore_axis_name="s")

    def body(table, src, dst2d, w, out, acc, zbuf, src_b, gidx_b, dst_b,
             w_b, rows, gsem):
        c = lax.axis_index("c")
        s = lax.axis_index("s")

        zero16 = jnp.zeros((LANES,), jnp.float32)
        for i in range(ZR):
            zbuf[i, 0, :] = zero16
        base_row = s * ROWS_PER_TILE

        def zero_body(k, carry):
            pltpu.sync_copy(zbuf, acc.at[pl.ds(base_row + k * ZR, ZR)])
            return carry

        lax.fori_loop(0, ROWS_PER_TILE // ZR, zero_body, 0)
        plsc.subcore_barrier()

        ebase = s * (nblk * BLOCK)
        cbase = s * (nblk * SUB)

        def block_body(b, carry):
            eoff = ebase + b * BLOCK
            pltpu.sync_copy(src.at[pl.ds(eoff, BLOCK)], src_b)
            pltpu.sync_copy(w.at[pl.ds(eoff, BLOCK)], w_b)
            pltpu.sync_copy(dst2d.at[pl.ds(cbase + b * SUB, SUB)], dst_b)

            for q in range(BLOCK // LANES):
                sv = src_b[pl.ds(q * LANES, LANES)]
                gidx_b[q // (CHUNK // LANES),
                       pl.ds((q % (CHUNK // LANES)) * LANES, LANES)] = (
                           sv * 2 + c)

            def sub_body(j, icarry):
                pltpu.async_copy(table.at[gidx_b.at[j]], rows, gsem).wait()
                for q in range(CHUNK // LANES):
                    w16 = w_b[pl.ds(j * CHUNK + q * LANES, LANES)]
                    for e in range(LANES):
                        r = q * LANES + e
                        wv = lax.broadcast(w16[e], (LANES,))
                        rows[r, 0, :] = rows[r, 0, :] * wv
                pltpu.sync_copy(rows, acc.at[dst_b.at[j]], add=True)
                return icarry

            lax.fori_loop(0, SUB, sub_body, 0)
            return carry

        lax.fori_loop(0, nblk, block_body, 0)
        plsc.subcore_barrier()

        pltpu.sync_copy(
            acc.at[pl.ds(base_row, ROWS_PER_TILE)],
            out.at[pl.ds(base_row, ROWS_PER_TILE), pl.ds(c, 1)])

    return pl.kernel(
        body,
        out_type=jax.ShapeDtypeStruct((NN, NC, HALF), jnp.float32),
        mesh=mesh,
        compiler_params=pltpu.CompilerParams(use_tc_tiling_on_sc=False),
        scratch_types=[
            pltpu.VMEM_SHARED((NN, 1, HALF), jnp.float32),
            pltpu.VMEM((ZR, 1, HALF), jnp.float32),
            pltpu.VMEM((BLOCK,), jnp.int32),
            pltpu.VMEM((SUB, CHUNK), jnp.int32),
            pltpu.VMEM((SUB, CHUNK), jnp.int32),
            pltpu.VMEM((BLOCK,), jnp.float32),
            pltpu.VMEM((CHUNK, 1, HALF), jnp.float32),
            pltpu.SemaphoreType.DMA,
        ],
    )


def _mean_body(e0, e1, e2, e3, out, b0, b1, b2, b3, ob):
    c = lax.axis_index("c")
    s = lax.axis_index("s")
    wid = s * NC + c
    base = wid * PER_W

    def chunk_body(k, carry):
        off = base + k * MEAN_CH
        pltpu.sync_copy(e0.at[pl.ds(off, MEAN_CH)], b0)
        pltpu.sync_copy(e1.at[pl.ds(off, MEAN_CH)], b1)
        pltpu.sync_copy(e2.at[pl.ds(off, MEAN_CH)], b2)
        pltpu.sync_copy(e3.at[pl.ds(off, MEAN_CH)], b3)
        for q in range(MEAN_CH // LANES):
            sl = pl.ds(q * LANES, LANES)
            ob[sl] = (b0[sl] + b1[sl] + b2[sl] + b3[sl]) * 0.25
        pltpu.sync_copy(ob, out.at[pl.ds(off, MEAN_CH)])
        return carry

    lax.fori_loop(0, PER_W // MEAN_CH, chunk_body, 0)


_mean_fn = pl.kernel(
    _mean_body,
    out_type=jax.ShapeDtypeStruct((TOT,), jnp.float32),
    mesh=plsc.VectorSubcoreMesh(core_axis_name="c", subcore_axis_name="s"),
    compiler_params=pltpu.CompilerParams(use_tc_tiling_on_sc=False),
    scratch_types=[pltpu.VMEM((MEAN_CH,), jnp.float32) for _ in range(5)],
)


def kernel(user_emb, item_emb, edge_index, edge_weight):
    ego0 = jnp.concatenate([user_emb, item_emb], axis=0)
    src = edge_index[0]
    dst = edge_index[1]
    e = src.shape[0]
    nblk = -(-e // (NS * BLOCK))
    epad = nblk * NS * BLOCK
    pad = epad - e
    src_p = jnp.concatenate([src, jnp.zeros((pad,), jnp.int32)])
    dst_p = jnp.concatenate([dst, jnp.zeros((pad,), jnp.int32)])
    w_p = jnp.concatenate([edge_weight, jnp.zeros((pad,), jnp.float32)])
    dst2d = dst_p.reshape(-1, CHUNK)

    layer = _layer_fn(nblk)
    t0 = ego0.reshape(NN * 2, 1, HALF)
    t1 = layer(t0, src_p, dst2d, w_p)
    t2 = layer(t1.reshape(NN * 2, 1, HALF), src_p, dst2d, w_p)
    t3 = layer(t2.reshape(NN * 2, 1, HALF), src_p, dst2d, w_p)

    m = _mean_fn(ego0.reshape(TOT), t1.reshape(TOT),
                 t2.reshape(TOT), t3.reshape(TOT))
    m = m.reshape(NN, DD)
    return (m[:USER_N], m[USER_N:])


# speedup vs baseline: 8.4586x; 1.3582x over previous
import functools

import jax
import jax.numpy as jnp
from jax import lax
from jax.experimental import pallas as pl
from jax.experimental.pallas import tpu as pltpu
from jax.experimental.pallas import tpu_sc as plsc

USER_N = 50000
ITEM_N = 50000
NN = USER_N + ITEM_N
DD = 32
HALF = 16
NC = 2
NS = 16
LANES = 16

CHUNK = 128
SUB = 8
BLOCK = CHUNK * SUB
ROWS_PER_TILE = NN // NS
ZR = 125

MEAN_CH = 4000
TOT = NN * DD
PER_W = TOT // (NC * NS)


@functools.lru_cache(maxsize=None)
def _layer_fn(nblk):
    mesh = plsc.VectorSubcoreMesh(core_axis_name="c", subcore_axis_name="s")

    def body(table, src, dst2d, w, out, acc, zbuf, src_b, gidx_b, dst_b,
             w_b, rows, gsem):
        c = lax.axis_index("c")
        s = lax.axis_index("s")

        zero16 = jnp.zeros((LANES,), jnp.float32)
        for i in range(ZR):
            zbuf[i, 0, :] = zero16
        base_row = s * ROWS_PER_TILE

        def zero_body(k, carry):
            pltpu.sync_copy(zbuf, acc.at[pl.ds(base_row + k * ZR, ZR)])
            return carry

        lax.fori_loop(0, ROWS_PER_TILE // ZR, zero_body, 0)
        plsc.subcore_barrier()

        ebase = s * (nblk * BLOCK)
        cbase = s * (nblk * SUB)

        def block_body(b, carry):
            eoff = ebase + b * BLOCK
            pltpu.sync_copy(src.at[pl.ds(eoff, BLOCK)], src_b)
            pltpu.sync_copy(w.at[pl.ds(eoff, BLOCK)], w_b)
            pltpu.sync_copy(dst2d.at[pl.ds(cbase + b * SUB, SUB)], dst_b)

            for q in range(BLOCK // LANES):
                sv = src_b[pl.ds(q * LANES, LANES)]
                gidx_b[q // (CHUNK // LANES),
                       pl.ds((q % (CHUNK // LANES)) * LANES, LANES)] = (
                           sv * 2 + c)

            gds = [pltpu.async_copy(table.at[gidx_b.at[j]], rows.at[j], gsem)
                   for j in range(SUB)]
            for d in gds:
                d.wait()

            def mul_body(j, icarry):
                for q in range(CHUNK // LANES):
                    w16 = w_b[pl.ds(j * CHUNK + q * LANES, LANES)]
                    for e in range(LANES):
                        r = q * LANES + e
                        wv = lax.broadcast(w16[e], (LANES,))
                        rows[j, r, 0, :] = rows[j, r, 0, :] * wv
                return icarry

            lax.fori_loop(0, SUB, mul_body, 0)

            sds = [pltpu.async_copy(rows.at[j], acc.at[dst_b.at[j]], gsem,
                                    add=True)
                   for j in range(SUB)]
            for d in sds:
                d.wait()
            return carry

        lax.fori_loop(0, nblk, block_body, 0)
        plsc.subcore_barrier()

        pltpu.sync_copy(
            acc.at[pl.ds(base_row, ROWS_PER_TILE)],
            out.at[pl.ds(base_row, ROWS_PER_TILE), pl.ds(c, 1)])

    return pl.kernel(
        body,
        out_type=jax.ShapeDtypeStruct((NN, NC, HALF), jnp.float32),
        mesh=mesh,
        compiler_params=pltpu.CompilerParams(use_tc_tiling_on_sc=False),
        scratch_types=[
            pltpu.VMEM_SHARED((NN, 1, HALF), jnp.float32),
            pltpu.VMEM((ZR, 1, HALF), jnp.float32),
            pltpu.VMEM((BLOCK,), jnp.int32),
            pltpu.VMEM((SUB, CHUNK), jnp.int32),
            pltpu.VMEM((SUB, CHUNK), jnp.int32),
            pltpu.VMEM((BLOCK,), jnp.float32),
            pltpu.VMEM((SUB, CHUNK, 1, HALF), jnp.float32),
            pltpu.SemaphoreType.DMA,
        ],
    )


def _mean_body(e0, e1, e2, e3, out, b0, b1, b2, b3, ob):
    c = lax.axis_index("c")
    s = lax.axis_index("s")
    wid = s * NC + c
    base = wid * PER_W

    def chunk_body(k, carry):
        off = base + k * MEAN_CH
        pltpu.sync_copy(e0.at[pl.ds(off, MEAN_CH)], b0)
        pltpu.sync_copy(e1.at[pl.ds(off, MEAN_CH)], b1)
        pltpu.sync_copy(e2.at[pl.ds(off, MEAN_CH)], b2)
        pltpu.sync_copy(e3.at[pl.ds(off, MEAN_CH)], b3)
        for q in range(MEAN_CH // LANES):
            sl = pl.ds(q * LANES, LANES)
            ob[sl] = (b0[sl] + b1[sl] + b2[sl] + b3[sl]) * 0.25
        pltpu.sync_copy(ob, out.at[pl.ds(off, MEAN_CH)])
        return carry

    lax.fori_loop(0, PER_W // MEAN_CH, chunk_body, 0)


_mean_fn = pl.kernel(
    _mean_body,
    out_type=jax.ShapeDtypeStruct((TOT,), jnp.float32),
    mesh=plsc.VectorSubcoreMesh(core_axis_name="c", subcore_axis_name="s"),
    compiler_params=pltpu.CompilerParams(use_tc_tiling_on_sc=False),
    scratch_types=[pltpu.VMEM((MEAN_CH,), jnp.float32) for _ in range(5)],
)


def kernel(user_emb, item_emb, edge_index, edge_weight):
    ego0 = jnp.concatenate([user_emb, item_emb], axis=0)
    src = edge_index[0]
    dst = edge_index[1]
    e = src.shape[0]
    nblk = -(-e // (NS * BLOCK))
    epad = nblk * NS * BLOCK
    pad = epad - e
    src_p = jnp.concatenate([src, jnp.zeros((pad,), jnp.int32)])
    dst_p = jnp.concatenate([dst, jnp.zeros((pad,), jnp.int32)])
    w_p = jnp.concatenate([edge_weight, jnp.zeros((pad,), jnp.float32)])
    dst2d = dst_p.reshape(-1, CHUNK)

    layer = _layer_fn(nblk)
    t0 = ego0.reshape(NN * 2, 1, HALF)
    t1 = layer(t0, src_p, dst2d, w_p)
    t2 = layer(t1.reshape(NN * 2, 1, HALF), src_p, dst2d, w_p)
    t3 = layer(t2.reshape(NN * 2, 1, HALF), src_p, dst2d, w_p)

    m = _mean_fn(ego0.reshape(TOT), t1.reshape(TOT),
                 t2.reshape(TOT), t3.reshape(TOT))
    m = m.reshape(NN, DD)
    return (m[:USER_N], m[USER_N:])
